# baseline (speedup 1.0000x reference)
import functools

import jax
import jax.numpy as jnp
from jax import lax
from jax.experimental import pallas as pl
from jax.experimental.pallas import tpu as pltpu
from jax.experimental.pallas import tpu_sc as plsc

N = 10000
D = 128
E = 320000
NG = 64
NCLS = 16

NCORE = 2
NSUB = 16
NW = NCORE * NSUB
B = 120
KTOT = 167
K0 = 113
K1 = KTOT - K0
CH0 = NSUB * K0
KD0 = 84
KD1 = KTOT - KD0
CHD0 = NSUB * KD0
TOTCH = NSUB * KTOT
EPAD = TOTCH * B
TRASH = N
ACC_ROWS = 10240
SROWS = ACC_ROWS // NSUB

RB = 400
NBLK = N // RB

_mesh = plsc.VectorSubcoreMesh(core_axis_name="c", subcore_axis_name="s",
                               num_cores=NCORE, num_subcores=NSUB)


@functools.partial(
    pl.kernel,
    out_type=jax.ShapeDtypeStruct((NCORE, ACC_ROWS, D), jnp.float32),
    mesh=_mesh,
    scratch_types=[
        pltpu.VMEM((B, D), jnp.float32),
        pltpu.VMEM((B, D), jnp.float32),
        pltpu.VMEM((B, D), jnp.float32),
        pltpu.VMEM((B,), jnp.int32),
        pltpu.VMEM((B,), jnp.int32),
        pltpu.VMEM((B,), jnp.int32),
        pltpu.VMEM((B,), jnp.int32),
        pltpu.VMEM((B,), jnp.int32),
        pltpu.VMEM((B,), jnp.int32),
        pltpu.VMEM_SHARED((ACC_ROWS, D), jnp.float32),
        pltpu.SemaphoreType.DMA,
        pltpu.SemaphoreType.DMA,
        pltpu.SemaphoreType.DMA,
        pltpu.SemaphoreType.DMA,
        pltpu.SemaphoreType.DMA,
        pltpu.SemaphoreType.DMA,
        pltpu.SemaphoreType.DMA,
    ],
)
def _sc_edge_scatter(xs_hbm, srcw_hbm, dstw_hbm, zrows_hbm, acc_hbm,
                     rows0, rows1, rows2, srci0, srci1, srci2,
                     dsti0, dsti1, dsti2, acc_sh,
                     gsem0, gsem1, gsem2, isem0, isem1, isem2, ssem):
    cid = lax.axis_index("c")
    sid = lax.axis_index("s")

    pltpu.sync_copy(zrows_hbm, acc_sh.at[pl.ds(sid * SROWS, SROWS)])
    plsc.subcore_barrier()

    base = jnp.where(cid == 0, sid * K0, CH0 + sid * K1)
    n = jnp.where(cid == 0, K0, K1)

    rows = (rows0, rows1, rows2)
    srci = (srci0, srci1, srci2)
    dsti = (dsti0, dsti1, dsti2)
    gsem = (gsem0, gsem1, gsem2)
    isem = (isem0, isem1, isem2)

    def idx_start(c, q):
        pltpu.async_copy(srcw_hbm.at[base + c], srci[q], isem[q])
        pltpu.async_copy(dstw_hbm.at[base + c], dsti[q], isem[q])

    def idx_wait(c, q):
        pltpu.make_async_copy(srcw_hbm.at[base + c], srci[q], isem[q]).wait()
        pltpu.make_async_copy(dstw_hbm.at[base + c], dsti[q], isem[q]).wait()

    idx_start(0, 0)
    idx_start(1, 1)
    idx_wait(0, 0)
    pltpu.async_copy(xs_hbm.at[srci[0]], rows0, gsem0)
    idx_wait(1, 1)
    pltpu.async_copy(xs_hbm.at[srci[1]], rows1, gsem1)

    @pl.when(n > 2)
    def _():
        idx_start(2, 2)

    def chunk(j, _):
        for p in range(3):
            jj = 3 * j + p
            q2 = (p + 2) % 3

            @pl.when(jj + 2 < n)
            def _():
                idx_wait(jj + 2, q2)
                pltpu.async_copy(xs_hbm.at[srci[q2]], rows[q2], gsem[q2])

            @pl.when(jj < n)
            def _():
                pltpu.make_async_copy(xs_hbm.at[srci[p]], rows[p],
                                      gsem[p]).wait()
                pltpu.async_copy(rows[p], acc_sh.at[dsti[p]], ssem,
                                 add=True).wait()

            @pl.when(jj + 3 < n)
            def _():
                idx_start(jj + 3, p)
        return 0

    lax.fori_loop(0, (n + 2) // 3, chunk, 0, unroll=False)

    plsc.subcore_barrier()
    pltpu.sync_copy(acc_sh.at[pl.ds(sid * SROWS, SROWS)],
                    acc_hbm.at[cid].at[pl.ds(sid * SROWS, SROWS)])


@functools.partial(
    pl.kernel,
    out_type=jax.ShapeDtypeStruct((NCORE, ACC_ROWS, D), jnp.float32),
    mesh=_mesh,
    scratch_types=[
        pltpu.VMEM((B,), jnp.int32),
        pltpu.VMEM((B,), jnp.int32),
        pltpu.VMEM((B, D), jnp.float32),
        pltpu.VMEM_SHARED((ACC_ROWS, D), jnp.float32),
        pltpu.SemaphoreType.DMA,
        pltpu.SemaphoreType.DMA,
    ],
)
def _sc_degree(dstw_hbm, ones_hbm, zrows_hbm, deg_hbm, dsti0, dsti1, ones_v,
               acc_sh, isem, ssem):
    cid = lax.axis_index("c")
    sid = lax.axis_index("s")

    pltpu.sync_copy(ones_hbm, ones_v)
    pltpu.sync_copy(zrows_hbm, acc_sh.at[pl.ds(sid * SROWS, SROWS)])
    plsc.subcore_barrier()

    base = jnp.where(cid == 0, sid * KD0, CHD0 + sid * KD1)
    n = jnp.where(cid == 0, KD0, KD1)
    dsti = (dsti0, dsti1)

    pltpu.sync_copy(dstw_hbm.at[base], dsti0)

    def chunk(j, _):
        for p in range(2):
            jj = 2 * j + p

            @pl.when(jj + 1 < n)
            def _():
                pltpu.async_copy(dstw_hbm.at[base + jj + 1], dsti[1 - p],
                                 isem)

            @pl.when(jj < n)
            def _():
                pltpu.async_copy(ones_v, acc_sh.at[dsti[p]], ssem,
                                 add=True).wait()

            @pl.when(jj + 1 < n)
            def _():
                pltpu.make_async_copy(dstw_hbm.at[base + jj + 1],
                                      dsti[1 - p], isem).wait()
        return 0

    lax.fori_loop(0, (n + 1) // 2, chunk, 0, unroll=False)

    plsc.subcore_barrier()
    pltpu.sync_copy(acc_sh.at[pl.ds(sid * SROWS, SROWS)],
                    deg_hbm.at[cid].at[pl.ds(sid * SROWS, SROWS)])


def _row_spec():
    return pl.BlockSpec((RB, D), lambda i: (i, 0))


def _full_spec(shape):
    return pl.BlockSpec(shape, lambda i: tuple(0 for _ in shape))


def _p1_body(x_ref, w_ref, d0_ref, d1_ref, xs_ref, dinv_ref):
    d = d0_ref[:, :1] + d1_ref[:, :1] + 1.0
    dinv = lax.rsqrt(d)
    dinv_ref[...] = dinv
    xs_ref[...] = dinv * jnp.dot(x_ref[...], w_ref[...],
                                 preferred_element_type=jnp.float32)


_p1_call = pl.pallas_call(
    _p1_body,
    grid=(NBLK,),
    in_specs=[
        _row_spec(),
        _full_spec((D, D)),
        pl.BlockSpec((RB, D), lambda i: (i, 0)),
        pl.BlockSpec((RB, D), lambda i: (i, 0)),
    ],
    out_specs=[_row_spec(), pl.BlockSpec((RB, 1), lambda i: (i, 0))],
    out_shape=[
        jax.ShapeDtypeStruct((N, D), jnp.float32),
        jax.ShapeDtypeStruct((N, 1), jnp.float32),
    ],
)


def _mk_mid(has_extra, want_x):
    def body(*refs):
        if has_extra:
            (a0, a1, xs, dinv, b, extra, w), outs = refs[:7], refs[7:]
        else:
            (a0, a1, xs, dinv, b, w), outs = refs[:6], refs[6:]
            extra = None
        xl = jnp.maximum(
            dinv[...] * (a0[...] + a1[...] + xs[...]) + b[...], 0.0)
        conv_in = xl + extra[...] if extra is not None else xl
        xsn = dinv[...] * jnp.dot(conv_in, w[...],
                                  preferred_element_type=jnp.float32)
        if want_x:
            outs[0][...] = xl
            outs[1][...] = xsn
        else:
            outs[0][...] = xsn

    in_specs = [
        _row_spec(), _row_spec(),
        _row_spec(),
        pl.BlockSpec((RB, 1), lambda i: (i, 0)),
        _full_spec((1, D)),
    ]
    if has_extra:
        in_specs.append(_row_spec())
    in_specs.append(_full_spec((D, D)))

    out_specs = [_row_spec()] * (2 if want_x else 1)
    out_shape = [jax.ShapeDtypeStruct((N, D), jnp.float32)] * (2 if want_x else 1)
    return pl.pallas_call(
        body, grid=(NBLK,), in_specs=in_specs,
        out_specs=out_specs, out_shape=out_shape)


_mid_first = _mk_mid(False, True)
_mid = _mk_mid(True, True)
_mid_last = _mk_mid(True, False)


def _m6_body(a0, a1, xs, dinv, b, batch_ref, wl, bl, y_ref, sums, cnt):
    i = pl.program_id(0)

    @pl.when(i == 0)
    def _():
        sums[...] = jnp.zeros_like(sums)
        cnt[...] = jnp.zeros_like(cnt)

    x6 = dinv[...] * (a0[...] + a1[...] + xs[...]) + b[...]
    g = lax.broadcasted_iota(jnp.int32, (RB, NG), 1)
    mask = jnp.where(g == batch_ref[...], 1.0, 0.0)
    dn = (((0,), (0,)), ((), ()))
    sums[...] += lax.dot_general(mask, x6, dn,
                                 preferred_element_type=jnp.float32)
    cnt[...] += lax.dot_general(mask, jnp.ones((RB, 1), jnp.float32), dn,
                                preferred_element_type=jnp.float32)

    @pl.when(i == pl.num_programs(0) - 1)
    def _():
        pooled = sums[...] / jnp.maximum(cnt[...], 1.0)
        y_ref[...] = jnp.dot(pooled, wl[...],
                             preferred_element_type=jnp.float32) + bl[...]


_m6_call = pl.pallas_call(
    _m6_body,
    grid=(NBLK,),
    in_specs=[
        _row_spec(), _row_spec(), _row_spec(),
        pl.BlockSpec((RB, 1), lambda i: (i, 0)),
        _full_spec((1, D)),
        pl.BlockSpec((RB, 1), lambda i: (i, 0)),
        _full_spec((D, NCLS)),
        _full_spec((1, NCLS)),
    ],
    out_specs=pl.BlockSpec((NG, NCLS), lambda i: (0, 0)),
    out_shape=jax.ShapeDtypeStruct((NG, NCLS), jnp.float32),
    scratch_shapes=[
        pltpu.VMEM((NG, D), jnp.float32),
        pltpu.VMEM((NG, 1), jnp.float32),
    ],
)


def kernel(x, edge_index, batch, W1, b1, W2, b2, W3, b3, W4, b4, W5, b5,
           W6, b6, Wl, bl):
    src = edge_index[0].astype(jnp.int32)
    dst = edge_index[1].astype(jnp.int32)
    npad = EPAD - E
    srcw = jnp.concatenate([src, jnp.zeros((npad,), jnp.int32)]
                           ).reshape(TOTCH, B)
    dstw = jnp.concatenate([dst, jnp.full((npad,), TRASH, jnp.int32)]
                           ).reshape(TOTCH, B)
    batch_col = batch.astype(jnp.int32).reshape(N, 1)
    bs = [b.reshape(1, D) for b in (b1, b2, b3, b4, b5, b6)]
    bl2 = bl.reshape(1, NCLS)

    zrows = jnp.zeros((SROWS, D), jnp.float32)
    ones_bd = jnp.ones((B, D), jnp.float32)

    deg = _sc_degree(dstw, ones_bd, zrows)
    xs1, dinv = _p1_call(x, W1, deg[0, :N], deg[1, :N])

    acc = _sc_edge_scatter(xs1, srcw, dstw, zrows)
    x1, xs2 = _mid_first(acc[0, :N], acc[1, :N], xs1, dinv, bs[0], W2)
    acc = _sc_edge_scatter(xs2, srcw, dstw, zrows)
    x2, xs3 = _mid(acc[0, :N], acc[1, :N], xs2, dinv, bs[1], x1, W3)
    acc = _sc_edge_scatter(xs3, srcw, dstw, zrows)
    x3, xs4 = _mid(acc[0, :N], acc[1, :N], xs3, dinv, bs[2], x2, W4)
    acc = _sc_edge_scatter(xs4, srcw, dstw, zrows)
    x4, xs5 = _mid(acc[0, :N], acc[1, :N], xs4, dinv, bs[3], x3, W5)
    acc = _sc_edge_scatter(xs5, srcw, dstw, zrows)
    (xs6,) = _mid_last(acc[0, :N], acc[1, :N], xs5, dinv, bs[4], x4, W6)
    acc = _sc_edge_scatter(xs6, srcw, dstw, zrows)
    y = _m6_call(acc[0, :N], acc[1, :N], xs6, dinv, bs[5], batch_col, Wl, bl2)
    return y


# speedup vs baseline: 1.0759x; 1.0759x over previous
import functools

import jax
import jax.numpy as jnp
from jax import lax
from jax.experimental import pallas as pl
from jax.experimental.pallas import tpu as pltpu
from jax.experimental.pallas import tpu_sc as plsc

N = 10000
D = 128
E = 320000
NG = 64
NCLS = 16

NCORE = 2
NSUB = 16
NW = NCORE * NSUB
B = 120
KTOT = 167
K0 = 101
K1 = KTOT - K0
CH0 = NSUB * K0
KD0 = 84
KD1 = KTOT - KD0
CHD0 = NSUB * KD0
TOTCH = NSUB * KTOT
EPAD = TOTCH * B
TRASH = N
ACC_ROWS = 10240
SROWS = ACC_ROWS // NSUB

RB = 400
NBLK = N // RB

_mesh = plsc.VectorSubcoreMesh(core_axis_name="c", subcore_axis_name="s",
                               num_cores=NCORE, num_subcores=NSUB)


@functools.partial(
    pl.kernel,
    out_type=jax.ShapeDtypeStruct((NCORE, ACC_ROWS, D), jnp.float32),
    mesh=_mesh,
    scratch_types=[
        pltpu.VMEM((B, D), jnp.float32),
        pltpu.VMEM((B, D), jnp.float32),
        pltpu.VMEM((B, D), jnp.float32),
        pltpu.VMEM((B,), jnp.int32),
        pltpu.VMEM((B,), jnp.int32),
        pltpu.VMEM((B,), jnp.int32),
        pltpu.VMEM((B,), jnp.int32),
        pltpu.VMEM((B,), jnp.int32),
        pltpu.VMEM((B,), jnp.int32),
        pltpu.VMEM_SHARED((ACC_ROWS, D), jnp.float32),
        pltpu.SemaphoreType.DMA,
        pltpu.SemaphoreType.DMA,
        pltpu.SemaphoreType.DMA,
        pltpu.SemaphoreType.DMA,
        pltpu.SemaphoreType.DMA,
        pltpu.SemaphoreType.DMA,
        pltpu.SemaphoreType.DMA,
    ],
)
def _sc_edge_scatter(xs_hbm, srcw_hbm, dstw_hbm, zrows_hbm, acc_hbm,
                     rows0, rows1, rows2, srci0, srci1, srci2,
                     dsti0, dsti1, dsti2, acc_sh,
                     gsem0, gsem1, gsem2, isem0, isem1, isem2, ssem):
    cid = lax.axis_index("c")
    sid = lax.axis_index("s")

    pltpu.sync_copy(zrows_hbm, acc_sh.at[pl.ds(sid * SROWS, SROWS)])
    plsc.subcore_barrier()

    base = jnp.where(cid == 0, sid * K0, CH0 + sid * K1)
    n = jnp.where(cid == 0, K0, K1)

    rows = (rows0, rows1, rows2)
    srci = (srci0, srci1, srci2)
    dsti = (dsti0, dsti1, dsti2)
    gsem = (gsem0, gsem1, gsem2)
    isem = (isem0, isem1, isem2)

    def idx_start(c, q):
        pltpu.async_copy(srcw_hbm.at[base + c], srci[q], isem[q])
        pltpu.async_copy(dstw_hbm.at[base + c], dsti[q], isem[q])

    def idx_wait(c, q):
        pltpu.make_async_copy(srcw_hbm.at[base + c], srci[q], isem[q]).wait()
        pltpu.make_async_copy(dstw_hbm.at[base + c], dsti[q], isem[q]).wait()

    idx_start(0, 0)
    idx_start(1, 1)
    idx_wait(0, 0)
    pltpu.async_copy(xs_hbm.at[srci[0]], rows0, gsem0)
    idx_wait(1, 1)
    pltpu.async_copy(xs_hbm.at[srci[1]], rows1, gsem1)

    @pl.when(n > 2)
    def _():
        idx_start(2, 2)

    def chunk(j, _):
        for p in range(3):
            jj = 3 * j + p
            q2 = (p + 2) % 3

            @pl.when(jj + 2 < n)
            def _():
                idx_wait(jj + 2, q2)
                pltpu.async_copy(xs_hbm.at[srci[q2]], rows[q2], gsem[q2])

            @pl.when(jj < n)
            def _():
                pltpu.make_async_copy(xs_hbm.at[srci[p]], rows[p],
                                      gsem[p]).wait()
                pltpu.async_copy(rows[p], acc_sh.at[dsti[p]], ssem,
                                 add=True).wait()

            @pl.when(jj + 3 < n)
            def _():
                idx_start(jj + 3, p)
        return 0

    lax.fori_loop(0, (n + 2) // 3, chunk, 0, unroll=False)

    plsc.subcore_barrier()
    pltpu.sync_copy(acc_sh.at[pl.ds(sid * SROWS, SROWS)],
                    acc_hbm.at[cid].at[pl.ds(sid * SROWS, SROWS)])


@functools.partial(
    pl.kernel,
    out_type=jax.ShapeDtypeStruct((NCORE, ACC_ROWS, D), jnp.float32),
    mesh=_mesh,
    scratch_types=[
        pltpu.VMEM((B,), jnp.int32),
        pltpu.VMEM((B,), jnp.int32),
        pltpu.VMEM((B, D), jnp.float32),
        pltpu.VMEM_SHARED((ACC_ROWS, D), jnp.float32),
        pltpu.SemaphoreType.DMA,
        pltpu.SemaphoreType.DMA,
    ],
)
def _sc_degree(dstw_hbm, ones_hbm, zrows_hbm, deg_hbm, dsti0, dsti1, ones_v,
               acc_sh, isem, ssem):
    cid = lax.axis_index("c")
    sid = lax.axis_index("s")

    pltpu.sync_copy(ones_hbm, ones_v)
    pltpu.sync_copy(zrows_hbm, acc_sh.at[pl.ds(sid * SROWS, SROWS)])
    plsc.subcore_barrier()

    base = jnp.where(cid == 0, sid * KD0, CHD0 + sid * KD1)
    n = jnp.where(cid == 0, KD0, KD1)
    dsti = (dsti0, dsti1)

    pltpu.sync_copy(dstw_hbm.at[base], dsti0)

    def chunk(j, _):
        for p in range(2):
            jj = 2 * j + p

            @pl.when(jj + 1 < n)
            def _():
                pltpu.async_copy(dstw_hbm.at[base + jj + 1], dsti[1 - p],
                                 isem)

            @pl.when(jj < n)
            def _():
                pltpu.async_copy(ones_v, acc_sh.at[dsti[p]], ssem,
                                 add=True).wait()

            @pl.when(jj + 1 < n)
            def _():
                pltpu.make_async_copy(dstw_hbm.at[base + jj + 1],
                                      dsti[1 - p], isem).wait()
        return 0

    lax.fori_loop(0, (n + 1) // 2, chunk, 0, unroll=False)

    plsc.subcore_barrier()
    pltpu.sync_copy(acc_sh.at[pl.ds(sid * SROWS, SROWS)],
                    deg_hbm.at[cid].at[pl.ds(sid * SROWS, SROWS)])


def _row_spec():
    return pl.BlockSpec((RB, D), lambda i: (i, 0))


def _full_spec(shape):
    return pl.BlockSpec(shape, lambda i: tuple(0 for _ in shape))


def _p1_body(x_ref, w_ref, d0_ref, d1_ref, xs_ref, dinv_ref):
    d = d0_ref[:, :1] + d1_ref[:, :1] + 1.0
    dinv = lax.rsqrt(d)
    dinv_ref[...] = dinv
    xs_ref[...] = dinv * jnp.dot(x_ref[...], w_ref[...],
                                 preferred_element_type=jnp.float32)


_p1_call = pl.pallas_call(
    _p1_body,
    grid=(NBLK,),
    in_specs=[
        _row_spec(),
        _full_spec((D, D)),
        pl.BlockSpec((RB, D), lambda i: (i, 0)),
        pl.BlockSpec((RB, D), lambda i: (i, 0)),
    ],
    out_specs=[_row_spec(), pl.BlockSpec((RB, 1), lambda i: (i, 0))],
    out_shape=[
        jax.ShapeDtypeStruct((N, D), jnp.float32),
        jax.ShapeDtypeStruct((N, 1), jnp.float32),
    ],
)


def _mk_mid(has_extra, want_x):
    def body(*refs):
        if has_extra:
            (a0, a1, xs, dinv, b, extra, w), outs = refs[:7], refs[7:]
        else:
            (a0, a1, xs, dinv, b, w), outs = refs[:6], refs[6:]
            extra = None
        xl = jnp.maximum(
            dinv[...] * (a0[...] + a1[...] + xs[...]) + b[...], 0.0)
        conv_in = xl + extra[...] if extra is not None else xl
        xsn = dinv[...] * jnp.dot(conv_in, w[...],
                                  preferred_element_type=jnp.float32)
        if want_x:
            outs[0][...] = xl
            outs[1][...] = xsn
        else:
            outs[0][...] = xsn

    in_specs = [
        _row_spec(), _row_spec(),
        _row_spec(),
        pl.BlockSpec((RB, 1), lambda i: (i, 0)),
        _full_spec((1, D)),
    ]
    if has_extra:
        in_specs.append(_row_spec())
    in_specs.append(_full_spec((D, D)))

    out_specs = [_row_spec()] * (2 if want_x else 1)
    out_shape = [jax.ShapeDtypeStruct((N, D), jnp.float32)] * (2 if want_x else 1)
    return pl.pallas_call(
        body, grid=(NBLK,), in_specs=in_specs,
        out_specs=out_specs, out_shape=out_shape)


_mid_first = _mk_mid(False, True)
_mid = _mk_mid(True, True)
_mid_last = _mk_mid(True, False)


def _m6_body(a0, a1, xs, dinv, b, batch_ref, wl, bl, y_ref, sums, cnt):
    i = pl.program_id(0)

    @pl.when(i == 0)
    def _():
        sums[...] = jnp.zeros_like(sums)
        cnt[...] = jnp.zeros_like(cnt)

    x6 = dinv[...] * (a0[...] + a1[...] + xs[...]) + b[...]
    g = lax.broadcasted_iota(jnp.int32, (RB, NG), 1)
    mask = jnp.where(g == batch_ref[...], 1.0, 0.0)
    dn = (((0,), (0,)), ((), ()))
    sums[...] += lax.dot_general(mask, x6, dn,
                                 preferred_element_type=jnp.float32)
    cnt[...] += lax.dot_general(mask, jnp.ones((RB, 1), jnp.float32), dn,
                                preferred_element_type=jnp.float32)

    @pl.when(i == pl.num_programs(0) - 1)
    def _():
        pooled = sums[...] / jnp.maximum(cnt[...], 1.0)
        y_ref[...] = jnp.dot(pooled, wl[...],
                             preferred_element_type=jnp.float32) + bl[...]


_m6_call = pl.pallas_call(
    _m6_body,
    grid=(NBLK,),
    in_specs=[
        _row_spec(), _row_spec(), _row_spec(),
        pl.BlockSpec((RB, 1), lambda i: (i, 0)),
        _full_spec((1, D)),
        pl.BlockSpec((RB, 1), lambda i: (i, 0)),
        _full_spec((D, NCLS)),
        _full_spec((1, NCLS)),
    ],
    out_specs=pl.BlockSpec((NG, NCLS), lambda i: (0, 0)),
    out_shape=jax.ShapeDtypeStruct((NG, NCLS), jnp.float32),
    scratch_shapes=[
        pltpu.VMEM((NG, D), jnp.float32),
        pltpu.VMEM((NG, 1), jnp.float32),
    ],
)


def kernel(x, edge_index, batch, W1, b1, W2, b2, W3, b3, W4, b4, W5, b5,
           W6, b6, Wl, bl):
    src = edge_index[0].astype(jnp.int32)
    dst = edge_index[1].astype(jnp.int32)
    npad = EPAD - E
    srcw = jnp.concatenate([src, jnp.zeros((npad,), jnp.int32)]
                           ).reshape(TOTCH, B)
    dstw = jnp.concatenate([dst, jnp.full((npad,), TRASH, jnp.int32)]
                           ).reshape(TOTCH, B)
    batch_col = batch.astype(jnp.int32).reshape(N, 1)
    bs = [b.reshape(1, D) for b in (b1, b2, b3, b4, b5, b6)]
    bl2 = bl.reshape(1, NCLS)

    zrows = jnp.zeros((SROWS, D), jnp.float32)
    ones_bd = jnp.ones((B, D), jnp.float32)

    deg = _sc_degree(dstw, ones_bd, zrows)
    xs1, dinv = _p1_call(x, W1, deg[0, :N], deg[1, :N])

    acc = _sc_edge_scatter(xs1, srcw, dstw, zrows)
    x1, xs2 = _mid_first(acc[0, :N], acc[1, :N], xs1, dinv, bs[0], W2)
    acc = _sc_edge_scatter(xs2, srcw, dstw, zrows)
    x2, xs3 = _mid(acc[0, :N], acc[1, :N], xs2, dinv, bs[1], x1, W3)
    acc = _sc_edge_scatter(xs3, srcw, dstw, zrows)
    x3, xs4 = _mid(acc[0, :N], acc[1, :N], xs3, dinv, bs[2], x2, W4)
    acc = _sc_edge_scatter(xs4, srcw, dstw, zrows)
    x4, xs5 = _mid(acc[0, :N], acc[1, :N], xs4, dinv, bs[3], x3, W5)
    acc = _sc_edge_scatter(xs5, srcw, dstw, zrows)
    (xs6,) = _mid_last(acc[0, :N], acc[1, :N], xs5, dinv, bs[4], x4, W6)
    acc = _sc_edge_scatter(xs6, srcw, dstw, zrows)
    y = _m6_call(acc[0, :N], acc[1, :N], xs6, dinv, bs[5], batch_col, Wl, bl2)
    return y


# speedup vs baseline: 1.0994x; 1.0219x over previous
import functools

import jax
import jax.numpy as jnp
from jax import lax
from jax.experimental import pallas as pl
from jax.experimental.pallas import tpu as pltpu
from jax.experimental.pallas import tpu_sc as plsc

N = 10000
D = 128
E = 320000
NG = 64
NCLS = 16

NCORE = 2
NSUB = 16
NW = NCORE * NSUB
B = 120
KTOT = 167
K0 = 97
K1 = KTOT - K0
CH0 = NSUB * K0
KD0 = 84
KD1 = KTOT - KD0
CHD0 = NSUB * KD0
TOTCH = NSUB * KTOT
EPAD = TOTCH * B
TRASH = N
ACC_ROWS = 10240
SROWS = ACC_ROWS // NSUB

RB = 400
NBLK = N // RB

_mesh = plsc.VectorSubcoreMesh(core_axis_name="c", subcore_axis_name="s",
                               num_cores=NCORE, num_subcores=NSUB)


@functools.partial(
    pl.kernel,
    out_type=jax.ShapeDtypeStruct((NCORE, ACC_ROWS, D), jnp.float32),
    mesh=_mesh,
    scratch_types=[
        pltpu.VMEM((B, D), jnp.float32),
        pltpu.VMEM((B, D), jnp.float32),
        pltpu.VMEM((B, D), jnp.float32),
        pltpu.VMEM((B,), jnp.int32),
        pltpu.VMEM((B,), jnp.int32),
        pltpu.VMEM((B,), jnp.int32),
        pltpu.VMEM((B,), jnp.int32),
        pltpu.VMEM((B,), jnp.int32),
        pltpu.VMEM((B,), jnp.int32),
        pltpu.VMEM_SHARED((ACC_ROWS, D), jnp.float32),
        pltpu.SemaphoreType.DMA,
        pltpu.SemaphoreType.DMA,
        pltpu.SemaphoreType.DMA,
        pltpu.SemaphoreType.DMA,
        pltpu.SemaphoreType.DMA,
        pltpu.SemaphoreType.DMA,
        pltpu.SemaphoreType.DMA,
    ],
)
def _sc_edge_scatter(xs_hbm, srcw_hbm, dstw_hbm, zrows_hbm, acc_hbm,
                     rows0, rows1, rows2, srci0, srci1, srci2,
                     dsti0, dsti1, dsti2, acc_sh,
                     gsem0, gsem1, gsem2, isem0, isem1, isem2, ssem):
    cid = lax.axis_index("c")
    sid = lax.axis_index("s")

    pltpu.sync_copy(zrows_hbm, acc_sh.at[pl.ds(sid * SROWS, SROWS)])
    plsc.subcore_barrier()

    base = jnp.where(cid == 0, sid * K0, CH0 + sid * K1)
    n = jnp.where(cid == 0, K0, K1)

    rows = (rows0, rows1, rows2)
    srci = (srci0, srci1, srci2)
    dsti = (dsti0, dsti1, dsti2)
    gsem = (gsem0, gsem1, gsem2)
    isem = (isem0, isem1, isem2)

    def idx_start(c, q):
        pltpu.async_copy(srcw_hbm.at[base + c], srci[q], isem[q])
        pltpu.async_copy(dstw_hbm.at[base + c], dsti[q], isem[q])

    def idx_wait(c, q):
        pltpu.make_async_copy(srcw_hbm.at[base + c], srci[q], isem[q]).wait()
        pltpu.make_async_copy(dstw_hbm.at[base + c], dsti[q], isem[q]).wait()

    idx_start(0, 0)
    idx_start(1, 1)
    idx_wait(0, 0)
    pltpu.async_copy(xs_hbm.at[srci[0]], rows0, gsem0)
    idx_wait(1, 1)
    pltpu.async_copy(xs_hbm.at[srci[1]], rows1, gsem1)

    @pl.when(n > 2)
    def _():
        idx_start(2, 2)

    def chunk(j, _):
        for p in range(3):
            jj = 3 * j + p
            q2 = (p + 2) % 3

            @pl.when(jj + 2 < n)
            def _():
                idx_wait(jj + 2, q2)
                pltpu.async_copy(xs_hbm.at[srci[q2]], rows[q2], gsem[q2])

            @pl.when(jj < n)
            def _():
                pltpu.make_async_copy(xs_hbm.at[srci[p]], rows[p],
                                      gsem[p]).wait()
                pltpu.async_copy(rows[p], acc_sh.at[dsti[p]], ssem,
                                 add=True).wait()

            @pl.when(jj + 3 < n)
            def _():
                idx_start(jj + 3, p)
        return 0

    lax.fori_loop(0, (n + 2) // 3, chunk, 0, unroll=False)

    plsc.subcore_barrier()
    pltpu.sync_copy(acc_sh.at[pl.ds(sid * SROWS, SROWS)],
                    acc_hbm.at[cid].at[pl.ds(sid * SROWS, SROWS)])


@functools.partial(
    pl.kernel,
    out_type=jax.ShapeDtypeStruct((NCORE, ACC_ROWS, D), jnp.float32),
    mesh=_mesh,
    scratch_types=[
        pltpu.VMEM((B,), jnp.int32),
        pltpu.VMEM((B,), jnp.int32),
        pltpu.VMEM((B, D), jnp.float32),
        pltpu.VMEM_SHARED((ACC_ROWS, D), jnp.float32),
        pltpu.SemaphoreType.DMA,
        pltpu.SemaphoreType.DMA,
    ],
)
def _sc_degree(dstw_hbm, ones_hbm, zrows_hbm, deg_hbm, dsti0, dsti1, ones_v,
               acc_sh, isem, ssem):
    cid = lax.axis_index("c")
    sid = lax.axis_index("s")

    pltpu.sync_copy(ones_hbm, ones_v)
    pltpu.sync_copy(zrows_hbm, acc_sh.at[pl.ds(sid * SROWS, SROWS)])
    plsc.subcore_barrier()

    base = jnp.where(cid == 0, sid * KD0, CHD0 + sid * KD1)
    n = jnp.where(cid == 0, KD0, KD1)
    dsti = (dsti0, dsti1)

    pltpu.sync_copy(dstw_hbm.at[base], dsti0)

    def chunk(j, _):
        for p in range(2):
            jj = 2 * j + p

            @pl.when(jj + 1 < n)
            def _():
                pltpu.async_copy(dstw_hbm.at[base + jj + 1], dsti[1 - p],
                                 isem)

            @pl.when(jj < n)
            def _():
                pltpu.async_copy(ones_v, acc_sh.at[dsti[p]], ssem,
                                 add=True).wait()

            @pl.when(jj + 1 < n)
            def _():
                pltpu.make_async_copy(dstw_hbm.at[base + jj + 1],
                                      dsti[1 - p], isem).wait()
        return 0

    lax.fori_loop(0, (n + 1) // 2, chunk, 0, unroll=False)

    plsc.subcore_barrier()
    pltpu.sync_copy(acc_sh.at[pl.ds(sid * SROWS, SROWS)],
                    deg_hbm.at[cid].at[pl.ds(sid * SROWS, SROWS)])


def _row_spec():
    return pl.BlockSpec((RB, D), lambda i: (i, 0))


def _full_spec(shape):
    return pl.BlockSpec(shape, lambda i: tuple(0 for _ in shape))


def _p1_body(x_ref, w_ref, d0_ref, d1_ref, xs_ref, dinv_ref):
    d = d0_ref[:, :1] + d1_ref[:, :1] + 1.0
    dinv = lax.rsqrt(d)
    dinv_ref[...] = dinv
    xs_ref[...] = dinv * jnp.dot(x_ref[...], w_ref[...],
                                 preferred_element_type=jnp.float32)


_p1_call = pl.pallas_call(
    _p1_body,
    grid=(NBLK,),
    in_specs=[
        _row_spec(),
        _full_spec((D, D)),
        pl.BlockSpec((RB, D), lambda i: (i, 0)),
        pl.BlockSpec((RB, D), lambda i: (i, 0)),
    ],
    out_specs=[_row_spec(), pl.BlockSpec((RB, 1), lambda i: (i, 0))],
    out_shape=[
        jax.ShapeDtypeStruct((N, D), jnp.float32),
        jax.ShapeDtypeStruct((N, 1), jnp.float32),
    ],
)


def _mk_mid(has_extra, want_x):
    def body(*refs):
        if has_extra:
            (a0, a1, xs, dinv, b, extra, w), outs = refs[:7], refs[7:]
        else:
            (a0, a1, xs, dinv, b, w), outs = refs[:6], refs[6:]
            extra = None
        xl = jnp.maximum(
            dinv[...] * (a0[...] + a1[...] + xs[...]) + b[...], 0.0)
        conv_in = xl + extra[...] if extra is not None else xl
        xsn = dinv[...] * jnp.dot(conv_in, w[...],
                                  preferred_element_type=jnp.float32)
        if want_x:
            outs[0][...] = xl
            outs[1][...] = xsn
        else:
            outs[0][...] = xsn

    in_specs = [
        _row_spec(), _row_spec(),
        _row_spec(),
        pl.BlockSpec((RB, 1), lambda i: (i, 0)),
        _full_spec((1, D)),
    ]
    if has_extra:
        in_specs.append(_row_spec())
    in_specs.append(_full_spec((D, D)))

    out_specs = [_row_spec()] * (2 if want_x else 1)
    out_shape = [jax.ShapeDtypeStruct((N, D), jnp.float32)] * (2 if want_x else 1)
    return pl.pallas_call(
        body, grid=(NBLK,), in_specs=in_specs,
        out_specs=out_specs, out_shape=out_shape)


_mid_first = _mk_mid(False, True)
_mid = _mk_mid(True, True)
_mid_last = _mk_mid(True, False)


def _m6_body(a0, a1, xs, dinv, b, batch_ref, wl, bl, y_ref, sums, cnt):
    i = pl.program_id(0)

    @pl.when(i == 0)
    def _():
        sums[...] = jnp.zeros_like(sums)
        cnt[...] = jnp.zeros_like(cnt)

    x6 = dinv[...] * (a0[...] + a1[...] + xs[...]) + b[...]
    g = lax.broadcasted_iota(jnp.int32, (RB, NG), 1)
    mask = jnp.where(g == batch_ref[...], 1.0, 0.0)
    dn = (((0,), (0,)), ((), ()))
    sums[...] += lax.dot_general(mask, x6, dn,
                                 preferred_element_type=jnp.float32)
    cnt[...] += lax.dot_general(mask, jnp.ones((RB, 1), jnp.float32), dn,
                                preferred_element_type=jnp.float32)

    @pl.when(i == pl.num_programs(0) - 1)
    def _():
        pooled = sums[...] / jnp.maximum(cnt[...], 1.0)
        y_ref[...] = jnp.dot(pooled, wl[...],
                             preferred_element_type=jnp.float32) + bl[...]


_m6_call = pl.pallas_call(
    _m6_body,
    grid=(NBLK,),
    in_specs=[
        _row_spec(), _row_spec(), _row_spec(),
        pl.BlockSpec((RB, 1), lambda i: (i, 0)),
        _full_spec((1, D)),
        pl.BlockSpec((RB, 1), lambda i: (i, 0)),
        _full_spec((D, NCLS)),
        _full_spec((1, NCLS)),
    ],
    out_specs=pl.BlockSpec((NG, NCLS), lambda i: (0, 0)),
    out_shape=jax.ShapeDtypeStruct((NG, NCLS), jnp.float32),
    scratch_shapes=[
        pltpu.VMEM((NG, D), jnp.float32),
        pltpu.VMEM((NG, 1), jnp.float32),
    ],
)


def kernel(x, edge_index, batch, W1, b1, W2, b2, W3, b3, W4, b4, W5, b5,
           W6, b6, Wl, bl):
    src = edge_index[0].astype(jnp.int32)
    dst = edge_index[1].astype(jnp.int32)
    npad = EPAD - E
    srcw = jnp.concatenate([src, jnp.zeros((npad,), jnp.int32)]
                           ).reshape(TOTCH, B)
    dstw = jnp.concatenate([dst, jnp.full((npad,), TRASH, jnp.int32)]
                           ).reshape(TOTCH, B)
    batch_col = batch.astype(jnp.int32).reshape(N, 1)
    bs = [b.reshape(1, D) for b in (b1, b2, b3, b4, b5, b6)]
    bl2 = bl.reshape(1, NCLS)

    zrows = jnp.zeros((SROWS, D), jnp.float32)
    ones_bd = jnp.ones((B, D), jnp.float32)

    deg = _sc_degree(dstw, ones_bd, zrows)
    xs1, dinv = _p1_call(x, W1, deg[0, :N], deg[1, :N])

    acc = _sc_edge_scatter(xs1, srcw, dstw, zrows)
    x1, xs2 = _mid_first(acc[0, :N], acc[1, :N], xs1, dinv, bs[0], W2)
    acc = _sc_edge_scatter(xs2, srcw, dstw, zrows)
    x2, xs3 = _mid(acc[0, :N], acc[1, :N], xs2, dinv, bs[1], x1, W3)
    acc = _sc_edge_scatter(xs3, srcw, dstw, zrows)
    x3, xs4 = _mid(acc[0, :N], acc[1, :N], xs3, dinv, bs[2], x2, W4)
    acc = _sc_edge_scatter(xs4, srcw, dstw, zrows)
    x4, xs5 = _mid(acc[0, :N], acc[1, :N], xs4, dinv, bs[3], x3, W5)
    acc = _sc_edge_scatter(xs5, srcw, dstw, zrows)
    (xs6,) = _mid_last(acc[0, :N], acc[1, :N], xs5, dinv, bs[4], x4, W6)
    acc = _sc_edge_scatter(xs6, srcw, dstw, zrows)
    y = _m6_call(acc[0, :N], acc[1, :N], xs6, dinv, bs[5], batch_col, Wl, bl2)
    return y
